# baseline (speedup 1.0000x reference)
import functools

import jax
import jax.numpy as jnp
from jax import lax
from jax.experimental import pallas as pl
from jax.experimental.pallas import tpu as pltpu
from jax.experimental.pallas import tpu_sc as plsc

N = 100000
D = 128
M = 25000

NUM_CORES = 2
NUM_SUBCORES = 16
NW = NUM_CORES * NUM_SUBCORES
B_PER_W = 392
HALF = B_PER_W * NW
M_PAD = 2 * HALF
CHUNKS = (112, 112, 112, 56)

_MESH = plsc.VectorSubcoreMesh(core_axis_name="c", subcore_axis_name="s")


def _make_sc_gather(offset):
    @functools.partial(
        pl.kernel,
        out_type=jax.ShapeDtypeStruct((HALF, D), jnp.float32),
        mesh=_MESH,
        scratch_types=[
            pltpu.VMEM((B_PER_W,), jnp.int32),
            pltpu.VMEM((B_PER_W, D), jnp.float32),
            pltpu.SemaphoreType.DMA,
        ],
        name=f"sc_gather_{offset}",
    )
    def _sc_gather(feat_hbm, idx_hbm, out_hbm, idx_v, rows_v, sem):
        wid = lax.axis_index("s") * NUM_CORES + lax.axis_index("c")
        base = wid * B_PER_W
        pltpu.sync_copy(idx_hbm.at[pl.ds(offset + base, B_PER_W)], idx_v)
        copies = []
        pos = 0
        for c in CHUNKS:
            copies.append(
                pltpu.async_copy(
                    feat_hbm.at[idx_v.at[pl.ds(pos, c)]],
                    rows_v.at[pl.ds(pos, c)],
                    sem,
                )
            )
            pos += c
        for c in copies:
            c.wait()
        pltpu.sync_copy(rows_v, out_hbm.at[pl.ds(base, B_PER_W)])

    return _sc_gather


_sc_gather_a = _make_sc_gather(0)
_sc_gather_b = _make_sc_gather(HALF)

_TM = 3136


def _mm_body_a(g_ref, w_ref, b_ref, o_ref):
    o_ref[...] = (
        lax.dot_general(
            g_ref[...], w_ref[...], (((1,), (1,)), ((), ())),
            preferred_element_type=jnp.float32,
        )
        + b_ref[...]
    )


def _mm_body_b(prev_ref, g_ref, w_ref, b_ref, o_ref):
    del prev_ref
    o_ref[...] = (
        lax.dot_general(
            g_ref[...], w_ref[...], (((1,), (1,)), ((), ())),
            preferred_element_type=jnp.float32,
        )
        + b_ref[...]
    )


def _tc_matmul_a(g, w2, b2):
    return pl.pallas_call(
        _mm_body_a,
        grid=(HALF // _TM,),
        in_specs=[
            pl.BlockSpec((_TM, D), lambda i: (i, 0)),
            pl.BlockSpec((D, D), lambda i: (0, 0)),
            pl.BlockSpec((1, D), lambda i: (0, 0)),
        ],
        out_specs=pl.BlockSpec((_TM, D), lambda i: (i, 0)),
        out_shape=jax.ShapeDtypeStruct((M, D), jnp.float32),
    )(g, w2, b2)


def _tc_matmul_b(prev, g, w2, b2):
    nb = HALF // _TM
    return pl.pallas_call(
        _mm_body_b,
        grid=(nb,),
        in_specs=[
            pl.BlockSpec((8, D), lambda i: (0, 0)),
            pl.BlockSpec((_TM, D), lambda i: (i, 0)),
            pl.BlockSpec((D, D), lambda i: (0, 0)),
            pl.BlockSpec((1, D), lambda i: (0, 0)),
        ],
        out_specs=pl.BlockSpec((_TM, D), lambda i: (i + nb, 0)),
        out_shape=jax.ShapeDtypeStruct((M, D), jnp.float32),
        input_output_aliases={0: 0},
    )(prev, g, w2, b2)


def kernel(feat, gtensor, itensor, idi, W, b):
    del gtensor, itensor
    d_out = W.shape[0]
    d_in = W.shape[-1]
    idx_pad = jnp.concatenate([idi, jnp.arange(M_PAD - M, dtype=jnp.int32)])
    ga = _sc_gather_a(feat, idx_pad)
    gb = _sc_gather_b(feat, idx_pad)
    w2 = W.reshape(d_out, d_in)
    b2 = b.reshape(1, D)
    out_a = _tc_matmul_a(ga, w2, b2)
    return _tc_matmul_b(out_a, gb, w2, b2)


# speedup vs baseline: 1.0379x; 1.0379x over previous
import functools

import jax
import jax.numpy as jnp
from jax import lax
from jax.experimental import pallas as pl
from jax.experimental.pallas import tpu as pltpu
from jax.experimental.pallas import tpu_sc as plsc

N = 100000
D = 128
M = 25000

NUM_CORES = 2
NUM_SUBCORES = 16
NW = NUM_CORES * NUM_SUBCORES
B_PER_W = 784
M_PAD = B_PER_W * NW
CHUNK = 112
N_CHUNKS = B_PER_W // CHUNK

_MESH = plsc.VectorSubcoreMesh(core_axis_name="c", subcore_axis_name="s")


@functools.partial(
    pl.kernel,
    out_type=jax.ShapeDtypeStruct((M_PAD, D), jnp.float32),
    mesh=_MESH,
    scratch_types=[
        pltpu.VMEM((B_PER_W,), jnp.int32),
        pltpu.VMEM((B_PER_W, D), jnp.float32),
        pltpu.SemaphoreType.DMA,
    ],
)
def _sc_gather(feat_hbm, idx_hbm, out_hbm, idx_v, rows_v, sem):
    wid = lax.axis_index("s") * NUM_CORES + lax.axis_index("c")
    base = wid * B_PER_W
    pltpu.sync_copy(idx_hbm.at[pl.ds(base, B_PER_W)], idx_v)
    copies = []
    for j in range(N_CHUNKS):
        copies.append(
            pltpu.async_copy(
                feat_hbm.at[idx_v.at[pl.ds(j * CHUNK, CHUNK)]],
                rows_v.at[pl.ds(j * CHUNK, CHUNK)],
                sem,
            )
        )
    for c in copies:
        c.wait()
    pltpu.sync_copy(rows_v, out_hbm.at[pl.ds(base, B_PER_W)])


def _mm_body(g_ref, w_ref, b_ref, o_ref):
    o_ref[...] = (
        lax.dot_general(
            g_ref[...], w_ref[...], (((1,), (1,)), ((), ())),
            preferred_element_type=jnp.float32,
        )
        + b_ref[...]
    )


_TM = 3136


def _tc_matmul(gathered, w2, b2):
    return pl.pallas_call(
        _mm_body,
        grid=(M_PAD // _TM,),
        in_specs=[
            pl.BlockSpec((_TM, D), lambda i: (i, 0)),
            pl.BlockSpec((D, D), lambda i: (0, 0)),
            pl.BlockSpec((1, D), lambda i: (0, 0)),
        ],
        out_specs=pl.BlockSpec((_TM, D), lambda i: (i, 0)),
        out_shape=jax.ShapeDtypeStruct((M, D), jnp.float32),
    )(gathered, w2, b2)


def kernel(feat, gtensor, itensor, idi, W, b):
    del gtensor, itensor
    d_out = W.shape[0]
    d_in = W.shape[-1]
    idx_pad = jnp.concatenate(
        [idi, jnp.arange(M_PAD - M, dtype=jnp.int32)]
    )
    gathered = _sc_gather(feat, idx_pad)
    w2 = W.reshape(d_out, d_in)
    b2 = b.reshape(1, D)
    return _tc_matmul(gathered, w2, b2)


# speedup vs baseline: 1.0805x; 1.0410x over previous
import functools

import jax
import jax.numpy as jnp
from jax import lax
from jax.experimental import pallas as pl
from jax.experimental.pallas import tpu as pltpu
from jax.experimental.pallas import tpu_sc as plsc

N = 100000
D = 128
M = 25000

NUM_CORES = 2
NUM_SUBCORES = 16
NW = NUM_CORES * NUM_SUBCORES
B_PER_W = 784
M_PAD = B_PER_W * NW
CHUNK = 112
N_CHUNKS = B_PER_W // CHUNK

_MESH = plsc.VectorSubcoreMesh(core_axis_name="c", subcore_axis_name="s")


@functools.partial(
    pl.kernel,
    out_type=jax.ShapeDtypeStruct((M_PAD, D), jnp.float32),
    mesh=_MESH,
    scratch_types=[
        pltpu.VMEM((B_PER_W,), jnp.int32),
        pltpu.VMEM((B_PER_W, D), jnp.float32),
        pltpu.SemaphoreType.DMA,
    ],
)
def _sc_gather(feat_hbm, idx_hbm, out_hbm, idx_v, rows_v, sem):
    wid = lax.axis_index("s") * NUM_CORES + lax.axis_index("c")
    base = wid * B_PER_W
    pltpu.sync_copy(idx_hbm.at[pl.ds(base, B_PER_W)], idx_v)
    copies = []
    for j in range(N_CHUNKS):
        copies.append(
            pltpu.async_copy(
                feat_hbm.at[idx_v.at[pl.ds(j * CHUNK, CHUNK)]],
                rows_v.at[pl.ds(j * CHUNK, CHUNK)],
                sem,
            )
        )
    for c in copies:
        c.wait()
    pltpu.sync_copy(rows_v, out_hbm.at[pl.ds(base, B_PER_W)])


def _mm_body(g_ref, w_ref, b_ref, o_ref):
    o_ref[...] = (
        lax.dot_general(
            g_ref[...], w_ref[...], (((1,), (1,)), ((), ())),
            preferred_element_type=jnp.float32,
        )
        + b_ref[...]
    )


_TM = 6272


def _tc_matmul(gathered, w2, b2):
    return pl.pallas_call(
        _mm_body,
        grid=(M_PAD // _TM,),
        in_specs=[
            pl.BlockSpec((_TM, D), lambda i: (i, 0)),
            pl.BlockSpec((D, D), lambda i: (0, 0)),
            pl.BlockSpec((1, D), lambda i: (0, 0)),
        ],
        out_specs=pl.BlockSpec((_TM, D), lambda i: (i, 0)),
        out_shape=jax.ShapeDtypeStruct((M, D), jnp.float32),
    )(gathered, w2, b2)


def kernel(feat, gtensor, itensor, idi, W, b):
    del gtensor, itensor
    d_out = W.shape[0]
    d_in = W.shape[-1]
    idx_pad = jnp.concatenate(
        [idi, jnp.arange(M_PAD - M, dtype=jnp.int32)]
    )
    gathered = _sc_gather(feat, idx_pad)
    w2 = W.reshape(d_out, d_in)
    b2 = b.reshape(1, D)
    return _tc_matmul(gathered, w2, b2)


# speedup vs baseline: 1.1394x; 1.0545x over previous
import functools

import jax
import jax.numpy as jnp
from jax import lax
from jax.experimental import pallas as pl
from jax.experimental.pallas import tpu as pltpu
from jax.experimental.pallas import tpu_sc as plsc

N = 100000
D = 128
M = 25000

NUM_CORES = 2
NUM_SUBCORES = 16
NW = NUM_CORES * NUM_SUBCORES
B_PER_W = 784
M_PAD = B_PER_W * NW
CHUNK = 112
N_CHUNKS = B_PER_W // CHUNK

_MESH = plsc.VectorSubcoreMesh(core_axis_name="c", subcore_axis_name="s")


@functools.partial(
    pl.kernel,
    out_type=jax.ShapeDtypeStruct((M_PAD, D), jnp.float32),
    mesh=_MESH,
    scratch_types=[
        pltpu.VMEM((B_PER_W,), jnp.int32),
        pltpu.VMEM((B_PER_W, D), jnp.float32),
        pltpu.SemaphoreType.DMA,
    ],
)
def _sc_gather(feat_hbm, idx_hbm, out_hbm, idx_v, rows_v, sem):
    wid = lax.axis_index("s") * NUM_CORES + lax.axis_index("c")
    base = wid * B_PER_W
    pltpu.sync_copy(idx_hbm.at[pl.ds(base, B_PER_W)], idx_v)
    copies = []
    for j in range(N_CHUNKS):
        copies.append(
            pltpu.async_copy(
                feat_hbm.at[idx_v.at[pl.ds(j * CHUNK, CHUNK)]],
                rows_v.at[pl.ds(j * CHUNK, CHUNK)],
                sem,
            )
        )
    for c in copies:
        c.wait()
    pltpu.sync_copy(rows_v, out_hbm.at[pl.ds(base, B_PER_W)])


def _mm_body(g_ref, w_ref, b_ref, o_ref):
    o_ref[...] = (
        lax.dot_general(
            g_ref[...], w_ref[...], (((1,), (1,)), ((), ())),
            preferred_element_type=jnp.float32,
        )
        + b_ref[...]
    )


_TM = 12544


def _tc_matmul(gathered, w2, b2):
    return pl.pallas_call(
        _mm_body,
        grid=(M_PAD // _TM,),
        in_specs=[
            pl.BlockSpec((_TM, D), lambda i: (i, 0)),
            pl.BlockSpec((D, D), lambda i: (0, 0)),
            pl.BlockSpec((1, D), lambda i: (0, 0)),
        ],
        out_specs=pl.BlockSpec((_TM, D), lambda i: (i, 0)),
        out_shape=jax.ShapeDtypeStruct((M, D), jnp.float32),
    )(gathered, w2, b2)


def kernel(feat, gtensor, itensor, idi, W, b):
    del gtensor, itensor
    d_out = W.shape[0]
    d_in = W.shape[-1]
    idx_pad = jnp.concatenate(
        [idi, jnp.arange(M_PAD - M, dtype=jnp.int32)]
    )
    gathered = _sc_gather(feat, idx_pad)
    w2 = W.reshape(d_out, d_in)
    b2 = b.reshape(1, D)
    return _tc_matmul(gathered, w2, b2)
